# Initial kernel scaffold; baseline (speedup 1.0000x reference)
import jax
import jax.numpy as jnp
from jax.experimental import pallas as pl


def kernel(xy, gamma, beta, layer_idx):
    raise NotImplementedError("write your pallas kernel here")


# speedup vs baseline: 43.4841x; 43.4841x over previous
import functools

import jax
import jax.numpy as jnp
from jax import lax
from jax.experimental import pallas as pl
from jax.experimental.pallas import tpu as pltpu
from jax.experimental.pallas import tpu_sc as plsc

N = 1048576
GRID = 64
C = 64
NUM_CORES = 2
NUM_SUBCORES = 16
NTILES = NUM_CORES * NUM_SUBCORES
PTS_PER_TILE = N // NTILES
P = 128
CHUNKS = PTS_PER_TILE // P
LANES = 16


def _sc_body(x_hbm, y_hbm, tbl_hbm, g_hbm, b_hbm,
             x_v, y_v, wx_v, wy_v, idx_v, dst_v, g_v, b_v, sem):
    wid = lax.axis_index("s") * NUM_CORES + lax.axis_index("c")
    tile_base = wid * PTS_PER_TILE

    def chunk_body(ci, _):
        base = tile_base + ci * P
        pltpu.sync_copy(x_hbm.at[pl.ds(base, P)], x_v)
        pltpu.sync_copy(y_hbm.at[pl.ds(base, P)], y_v)

        for j in range(P // LANES):
            sl = pl.ds(j * LANES, LANES)
            xv = x_v[sl]
            yv = y_v[sl]
            ix = jnp.clip((xv + 1.0) * 0.5 * (GRID - 1), 0.0, float(GRID - 1))
            iy = jnp.clip((yv + 1.0) * 0.5 * (GRID - 1), 0.0, float(GRID - 1))
            x0 = jnp.minimum(ix.astype(jnp.int32), GRID - 2)
            y0 = jnp.minimum(iy.astype(jnp.int32), GRID - 2)
            wx_v[sl] = ix - x0.astype(jnp.float32)
            wy_v[sl] = iy - y0.astype(jnp.float32)
            r00 = y0 * GRID + x0
            idx_v[0, sl] = r00
            idx_v[1, sl] = r00 + 1
            idx_v[2, sl] = r00 + GRID
            idx_v[3, sl] = r00 + GRID + 1

        descs = [
            pltpu.async_copy(tbl_hbm.at[idx_v.at[k]], dst_v.at[k], sem)
            for k in range(4)
        ]
        for d in descs:
            d.wait()

        def point_body(i, _):
            wxl = wx_v[pl.ds(i, LANES)]
            wyl = wy_v[pl.ds(i, LANES)]
            wxv = jnp.full((LANES,), wxl[0], jnp.float32)
            wyv = jnp.full((LANES,), wyl[0], jnp.float32)
            for ch in range(2 * C // LANES):
                sl = pl.ds(ch * LANES, LANES)
                v00 = dst_v[0, i, sl]
                v01 = dst_v[1, i, sl]
                v10 = dst_v[2, i, sl]
                v11 = dst_v[3, i, sl]
                t0 = v00 + wxv * (v01 - v00)
                t1 = v10 + wxv * (v11 - v10)
                o = t0 + wyv * (t1 - t0)
                if ch < C // LANES:
                    g_v[i, sl] = o
                else:
                    b_v[i, pl.ds((ch - C // LANES) * LANES, LANES)] = o
            return _

        lax.fori_loop(0, P, point_body, None)

        pltpu.sync_copy(g_v, g_hbm.at[pl.ds(base, P)])
        pltpu.sync_copy(b_v, b_hbm.at[pl.ds(base, P)])
        return _

    lax.fori_loop(0, CHUNKS, chunk_body, None)


@jax.jit
def _sc_call(x, y, tbl):
    mesh = plsc.VectorSubcoreMesh(
        core_axis_name="c", subcore_axis_name="s",
        num_cores=NUM_CORES, num_subcores=NUM_SUBCORES)
    fn = pl.kernel(
        _sc_body,
        out_type=(
            jax.ShapeDtypeStruct((N, C), jnp.float32),
            jax.ShapeDtypeStruct((N, C), jnp.float32),
        ),
        mesh=mesh,
        scratch_types=[
            pltpu.VMEM((P,), jnp.float32),
            pltpu.VMEM((P,), jnp.float32),
            pltpu.VMEM((P + LANES,), jnp.float32),
            pltpu.VMEM((P + LANES,), jnp.float32),
            pltpu.VMEM((4, P), jnp.int32),
            pltpu.VMEM((4, P, 2 * C), jnp.float32),
            pltpu.VMEM((P, C), jnp.float32),
            pltpu.VMEM((P, C), jnp.float32),
            pltpu.SemaphoreType.DMA,
        ],
    )
    return fn(x, y, tbl)


def kernel(xy, gamma, beta, layer_idx):
    tab = jnp.concatenate([gamma[layer_idx], beta[layer_idx]], axis=0)
    tbl = tab.reshape(2 * C, GRID * GRID).T
    x = xy[:, 0]
    y = xy[:, 1]
    return _sc_call(x, y, tbl)
